# speedup vs baseline: 3.3017x; 3.3017x over previous
import functools

import jax
import jax.numpy as jnp
from jax import lax
from jax.experimental import pallas as pl
from jax.experimental.pallas import tpu as pltpu
from jax.experimental.pallas import tpu_sc as plsc

NUM_ROWS = 100000
DIM = 128
B = 4096 * 50
NW = 32
B_PER_W = B // NW
CHUNK = 640
NCHUNK = B_PER_W // CHUNK

_mesh = plsc.VectorSubcoreMesh(core_axis_name="c", subcore_axis_name="s")


@functools.partial(
    pl.kernel,
    mesh=_mesh,
    out_type=jax.ShapeDtypeStruct((B, DIM), jnp.float32),
    scratch_types=[
        pltpu.VMEM((B_PER_W,), jnp.int32),
        pltpu.VMEM((CHUNK, DIM), jnp.float32),
        pltpu.SemaphoreType.DMA,
    ],
)
def _gather_kernel(idx_hbm, table_hbm, out_hbm, idx_v, rows_v, sem):
    wid = lax.axis_index("s") * 2 + lax.axis_index("c")
    base = wid * B_PER_W
    pltpu.sync_copy(idx_hbm.at[pl.ds(base, B_PER_W)], idx_v)

    def body(g, carry):
        off = g * CHUNK
        pltpu.async_copy(
            table_hbm.at[idx_v.at[pl.ds(off, CHUNK)]], rows_v, sem
        ).wait()
        pltpu.sync_copy(rows_v, out_hbm.at[pl.ds(base + off, CHUNK)])
        return carry

    lax.fori_loop(0, NCHUNK, body, 0)


def kernel(ids, weight):
    flat_ids = ids.reshape(B).astype(jnp.int32)
    out = _gather_kernel(flat_ids, weight)
    return out.reshape(ids.shape[0], ids.shape[1], DIM)


# speedup vs baseline: 3.3424x; 1.0123x over previous
import functools

import jax
import jax.numpy as jnp
from jax import lax
from jax.experimental import pallas as pl
from jax.experimental.pallas import tpu as pltpu
from jax.experimental.pallas import tpu_sc as plsc

DIM = 128
B = 4096 * 50
NW = 32
B_PER_W = B // NW
CHUNK = 400
NCHUNK = B_PER_W // CHUNK

_mesh = plsc.VectorSubcoreMesh(core_axis_name="c", subcore_axis_name="s")


@functools.partial(
    pl.kernel,
    mesh=_mesh,
    out_type=jax.ShapeDtypeStruct((B, DIM), jnp.float32),
    scratch_types=[
        pltpu.VMEM((B_PER_W,), jnp.int32),
        pltpu.VMEM((CHUNK, DIM), jnp.float32),
        pltpu.VMEM((CHUNK, DIM), jnp.float32),
        pltpu.SemaphoreType.DMA,
        pltpu.SemaphoreType.DMA,
        pltpu.SemaphoreType.DMA,
        pltpu.SemaphoreType.DMA,
    ],
)
def _gather_kernel(idx_hbm, table_hbm, out_hbm, idx_v, rows0, rows1,
                   gs0, gs1, ss0, ss1):
    wid = lax.axis_index("s") * 2 + lax.axis_index("c")
    base = wid * B_PER_W
    pltpu.sync_copy(idx_hbm.at[pl.ds(base, B_PER_W)], idx_v)

    def gather(g, rows, sem):
        pltpu.async_copy(table_hbm.at[idx_v.at[pl.ds(g * CHUNK, CHUNK)]],
                         rows, sem)

    def store(g, rows, sem):
        pltpu.async_copy(rows, out_hbm.at[pl.ds(base + g * CHUNK, CHUNK)],
                         sem)

    def wait_gather(rows, sem):
        pltpu.make_async_copy(out_hbm.at[pl.ds(0, CHUNK)], rows, sem).wait()

    def wait_store(rows, sem):
        pltpu.make_async_copy(rows, out_hbm.at[pl.ds(0, CHUNK)], sem).wait()

    gather(0, rows0, gs0)
    gather(1, rows1, gs1)
    wait_gather(rows0, gs0)
    store(0, rows0, ss0)

    def body(t, carry):
        g = 2 * t
        wait_store(rows0, ss0)
        gather(g, rows0, gs0)
        wait_gather(rows1, gs1)
        store(g - 1, rows1, ss1)
        wait_store(rows1, ss1)
        gather(g + 1, rows1, gs1)
        wait_gather(rows0, gs0)
        store(g, rows0, ss0)
        return carry

    lax.fori_loop(1, NCHUNK // 2, body, 0)

    wait_gather(rows1, gs1)
    store(NCHUNK - 1, rows1, ss1)
    wait_store(rows0, ss0)
    wait_store(rows1, ss1)


def kernel(ids, weight):
    flat_ids = ids.reshape(B).astype(jnp.int32)
    out = _gather_kernel(flat_ids, weight)
    return out.reshape(ids.shape[0], ids.shape[1], DIM)
